# baseline (speedup 1.0000x reference)
import functools

import jax
import jax.numpy as jnp
from jax import lax
from jax.experimental import pallas as pl
from jax.experimental.pallas import tpu as pltpu
from jax.experimental.pallas import tpu_sc as plsc

N = 100000
D = 128
G = 1024
HPAD = G + 16
NC = 2
NS = 16
NW = NC * NS
BLK = 80
NBT = N // BLK
FULLIT = NBT // NW
NTAIL = NBT - FULLIT * NW
NBUF = 6
SDEPTH = 3
NGRP = FULLIT // NBUF
EPI = FULLIT - NGRP * NBUF
GPT = G // NS


def _sc_partials(x, ids):
    mesh = plsc.VectorSubcoreMesh(core_axis_name="c", subcore_axis_name="s")

    @functools.partial(
        pl.kernel,
        out_type=(
            jax.ShapeDtypeStruct((NC, G, D), jnp.float32),
            jax.ShapeDtypeStruct((NW * G,), jnp.int32),
        ),
        mesh=mesh,
        scratch_types=(
            [pltpu.VMEM((BLK,), jnp.int32) for _ in range(NBUF)]
            + [pltpu.VMEM((BLK, D), jnp.float32) for _ in range(NBUF)]
            + [
                pltpu.VMEM((HPAD,), jnp.int32),
                pltpu.VMEM_SHARED((G, D), jnp.float32),
                pltpu.VMEM((BLK,), jnp.int32),
                pltpu.VMEM((BLK, D), jnp.float32),
            ]
            + [pltpu.SemaphoreType.DMA for _ in range(3 * NBUF)]
        ),
    )
    def k(x_hbm, idx_hbm, osum, ocnt, *bufs):
        idxb = list(bufs[0:NBUF])
        rowb = list(bufs[NBUF:2 * NBUF])
        hist_v = bufs[2 * NBUF]
        acc_s = bufs[2 * NBUF + 1]
        zidx = bufs[2 * NBUF + 2]
        zrow = bufs[2 * NBUF + 3]
        si = list(bufs[2 * NBUF + 4:2 * NBUF + 4 + NBUF])
        sx = list(bufs[2 * NBUF + 4 + NBUF:2 * NBUF + 4 + 2 * NBUF])
        ss = list(bufs[2 * NBUF + 4 + 2 * NBUF:2 * NBUF + 4 + 3 * NBUF])
        c = lax.axis_index("c")
        s = lax.axis_index("s")
        wid = s * NC + c
        iota = lax.iota(jnp.int32, 16)
        inc16 = jnp.where(iota == 0, 16, 0).astype(jnp.int32)
        inc1 = jnp.where(iota == 0, 1, 0).astype(jnp.int32)
        zero16 = jnp.zeros((16,), jnp.int32)
        zero16f = jnp.zeros((16,), jnp.float32)

        def issue_gathers(it, b):
            blk = it * NW + wid
            pltpu.async_copy(idx_hbm.at[pl.ds(blk * BLK, BLK)], idxb[b], si[b])
            pltpu.async_copy(x_hbm.at[pl.ds(blk * BLK, BLK)], rowb[b], sx[b])

        def wait_gathers(b):
            pltpu.make_async_copy(
                idx_hbm.at[pl.ds(0, BLK)], idxb[b], si[b]).wait()
            pltpu.make_async_copy(
                x_hbm.at[pl.ds(0, BLK)], rowb[b], sx[b]).wait()

        def issue_scatter(b):
            pltpu.async_copy(rowb[b], acc_s.at[idxb[b]], ss[b], add=True)

        def wait_scatter(b):
            pltpu.make_async_copy(rowb[b], acc_s.at[idxb[b]], ss[b]).wait()

        def hist_update(idx_ref):
            for vi in range(BLK // 16):
                v = idx_ref[pl.ds(vi * 16, 16)]
                first = v[0]
                single_run = first == v[15]

                @pl.when(single_run)
                def _():
                    h = hist_v[pl.ds(first, 16)]
                    hist_v[pl.ds(first, 16)] = h + inc16

                @pl.when(jnp.logical_not(single_run))
                def _():
                    for l in range(16):
                        g = v[l]
                        h = hist_v[pl.ds(g, 16)]
                        hist_v[pl.ds(g, 16)] = h + inc1

        for i in range(HPAD // 16):
            hist_v[pl.ds(i * 16, 16)] = zero16
        for i in range(BLK // 16):
            zidx[pl.ds(i * 16, 16)] = zero16

        def zrow_body(r, carry):
            for q in range(D // 16):
                zrow[r, pl.ds(q * 16, 16)] = zero16f
            return carry

        lax.fori_loop(0, BLK, zrow_body, jnp.int32(0))
        pltpu.sync_copy(zrow.at[pl.ds(0, GPT)],
                        acc_s.at[pl.ds(s * GPT, GPT)])
        plsc.subcore_barrier()
        for d in range(1, SDEPTH):
            pltpu.async_copy(zrow, acc_s.at[zidx], ss[(-d) % NBUF], add=True)

        for b in range(NBUF):
            issue_gathers(jnp.int32(b), b)

        def drain_scatter_slot(pb):
            pltpu.make_async_copy(zrow, acc_s.at[zidx], ss[pb]).wait()

        def step(it, b):
            wait_gathers(b)
            issue_scatter(b)
            hist_update(idxb[b])
            pb = (b - (SDEPTH - 1)) % NBUF
            drain_scatter_slot(pb)
            nxt = it + NBUF - (SDEPTH - 1)

            @pl.when(jnp.logical_and(it >= SDEPTH - 1, nxt < FULLIT))
            def _():
                issue_gathers(nxt, pb)

        def ring_body(j, carry):
            for b in range(NBUF):
                step(NBUF * j + b, b)
            return carry

        lax.fori_loop(0, NGRP, ring_body, jnp.int32(0))

        for e in range(EPI):
            step(jnp.int32(NGRP * NBUF + e), e)
        for d in range(SDEPTH - 1, 0, -1):
            drain_scatter_slot((FULLIT - d) % NBUF)

        @pl.when(wid < NTAIL)
        def _():
            blk = FULLIT * NW + wid
            pltpu.sync_copy(idx_hbm.at[pl.ds(blk * BLK, BLK)], idxb[0])
            pltpu.sync_copy(x_hbm.at[pl.ds(blk * BLK, BLK)], rowb[0])
            pltpu.sync_copy(rowb[0], acc_s.at[idxb[0]], add=True)
            hist_update(idxb[0])

        plsc.subcore_barrier()
        pltpu.sync_copy(acc_s.at[pl.ds(s * GPT, GPT)],
                        osum.at[c, pl.ds(s * GPT, GPT)])
        pltpu.sync_copy(hist_v.at[pl.ds(0, G)], ocnt.at[pl.ds(wid * G, G)])

    return k(x, ids)


def _combine(psum, pcnt):
    def body(ps_ref, pc_ref, o_ref):
        ps = ps_ref[...]
        cnt = jnp.sum(pc_ref[...].reshape(NW, G), axis=0).astype(jnp.float32)
        cnt = jnp.maximum(cnt, 1.0)
        o_ref[...] = (ps[0] + ps[1]) / cnt[:, None]

    return pl.pallas_call(
        body,
        out_shape=jax.ShapeDtypeStruct((G, D), jnp.float32),
    )(psum, pcnt)


def kernel(input, batch, num_graphs):
    ids = batch.astype(jnp.int32)
    psum, pcnt = _sc_partials(input, ids)
    return _combine(psum, pcnt)


# speedup vs baseline: 1.0456x; 1.0456x over previous
import functools

import jax
import jax.numpy as jnp
from jax import lax
from jax.experimental import pallas as pl
from jax.experimental.pallas import tpu as pltpu
from jax.experimental.pallas import tpu_sc as plsc

N = 100000
D = 128
G = 1024
HPAD = G + 16
NC = 2
NS = 16
NW = NC * NS
BLK = 80
NBT = N // BLK
FULLIT = NBT // NW
NTAIL = NBT - FULLIT * NW
NBUF = 4
SDEPTH = 2
NGRP = FULLIT // NBUF
EPI = FULLIT - NGRP * NBUF
GPT = G // NS


def _sc_partials(x, ids):
    mesh = plsc.VectorSubcoreMesh(core_axis_name="c", subcore_axis_name="s")

    @functools.partial(
        pl.kernel,
        out_type=(
            jax.ShapeDtypeStruct((NC, G, D), jnp.float32),
            jax.ShapeDtypeStruct((NW * G,), jnp.int32),
        ),
        mesh=mesh,
        scratch_types=(
            [pltpu.VMEM((BLK,), jnp.int32) for _ in range(NBUF)]
            + [pltpu.VMEM((BLK, D), jnp.float32) for _ in range(NBUF)]
            + [
                pltpu.VMEM((HPAD,), jnp.int32),
                pltpu.VMEM_SHARED((G, D), jnp.float32),
                pltpu.VMEM((BLK,), jnp.int32),
                pltpu.VMEM((BLK, D), jnp.float32),
            ]
            + [pltpu.SemaphoreType.DMA for _ in range(3 * NBUF)]
        ),
    )
    def k(x_hbm, idx_hbm, osum, ocnt, *bufs):
        idxb = list(bufs[0:NBUF])
        rowb = list(bufs[NBUF:2 * NBUF])
        hist_v = bufs[2 * NBUF]
        acc_s = bufs[2 * NBUF + 1]
        zidx = bufs[2 * NBUF + 2]
        zrow = bufs[2 * NBUF + 3]
        si = list(bufs[2 * NBUF + 4:2 * NBUF + 4 + NBUF])
        sx = list(bufs[2 * NBUF + 4 + NBUF:2 * NBUF + 4 + 2 * NBUF])
        ss = list(bufs[2 * NBUF + 4 + 2 * NBUF:2 * NBUF + 4 + 3 * NBUF])
        c = lax.axis_index("c")
        s = lax.axis_index("s")
        wid = s * NC + c
        iota = lax.iota(jnp.int32, 16)
        inc16 = jnp.where(iota == 0, 16, 0).astype(jnp.int32)
        inc1 = jnp.where(iota == 0, 1, 0).astype(jnp.int32)
        zero16 = jnp.zeros((16,), jnp.int32)
        zero16f = jnp.zeros((16,), jnp.float32)

        def issue_gathers(it, b):
            blk = it * NW + wid
            pltpu.async_copy(idx_hbm.at[pl.ds(blk * BLK, BLK)], idxb[b], si[b])
            pltpu.async_copy(x_hbm.at[pl.ds(blk * BLK, BLK)], rowb[b], sx[b])

        def wait_gathers(b):
            pltpu.make_async_copy(
                idx_hbm.at[pl.ds(0, BLK)], idxb[b], si[b]).wait()
            pltpu.make_async_copy(
                x_hbm.at[pl.ds(0, BLK)], rowb[b], sx[b]).wait()

        def issue_scatter(b):
            pltpu.async_copy(rowb[b], acc_s.at[idxb[b]], ss[b], add=True)

        def wait_scatter(b):
            pltpu.make_async_copy(rowb[b], acc_s.at[idxb[b]], ss[b]).wait()

        def hist_update(idx_ref):
            for vi in range(BLK // 16):
                v = idx_ref[pl.ds(vi * 16, 16)]
                first = v[0]
                single_run = first == v[15]

                @pl.when(single_run)
                def _():
                    h = hist_v[pl.ds(first, 16)]
                    hist_v[pl.ds(first, 16)] = h + inc16

                @pl.when(jnp.logical_not(single_run))
                def _():
                    for l in range(16):
                        g = v[l]
                        h = hist_v[pl.ds(g, 16)]
                        hist_v[pl.ds(g, 16)] = h + inc1

        for i in range(HPAD // 16):
            hist_v[pl.ds(i * 16, 16)] = zero16
        for i in range(BLK // 16):
            zidx[pl.ds(i * 16, 16)] = zero16

        def zrow_body(r, carry):
            for q in range(D // 16):
                zrow[r, pl.ds(q * 16, 16)] = zero16f
            return carry

        lax.fori_loop(0, BLK, zrow_body, jnp.int32(0))
        pltpu.sync_copy(zrow.at[pl.ds(0, GPT)],
                        acc_s.at[pl.ds(s * GPT, GPT)])
        plsc.subcore_barrier()
        for d in range(1, SDEPTH):
            pltpu.async_copy(zrow, acc_s.at[zidx], ss[(-d) % NBUF], add=True)

        for b in range(NBUF):
            issue_gathers(jnp.int32(b), b)

        def drain_scatter_slot(pb):
            pltpu.make_async_copy(zrow, acc_s.at[zidx], ss[pb]).wait()

        def step(it, b):
            wait_gathers(b)
            issue_scatter(b)
            hist_update(idxb[b])
            pb = (b - (SDEPTH - 1)) % NBUF
            drain_scatter_slot(pb)
            nxt = it + NBUF - (SDEPTH - 1)

            @pl.when(jnp.logical_and(it >= SDEPTH - 1, nxt < FULLIT))
            def _():
                issue_gathers(nxt, pb)

        def ring_body(j, carry):
            for b in range(NBUF):
                step(NBUF * j + b, b)
            return carry

        lax.fori_loop(0, NGRP, ring_body, jnp.int32(0))

        for e in range(EPI):
            step(jnp.int32(NGRP * NBUF + e), e)
        for d in range(SDEPTH - 1, 0, -1):
            drain_scatter_slot((FULLIT - d) % NBUF)

        @pl.when(wid < NTAIL)
        def _():
            blk = FULLIT * NW + wid
            pltpu.sync_copy(idx_hbm.at[pl.ds(blk * BLK, BLK)], idxb[0])
            pltpu.sync_copy(x_hbm.at[pl.ds(blk * BLK, BLK)], rowb[0])
            pltpu.sync_copy(rowb[0], acc_s.at[idxb[0]], add=True)
            hist_update(idxb[0])

        plsc.subcore_barrier()
        pltpu.sync_copy(acc_s.at[pl.ds(s * GPT, GPT)],
                        osum.at[c, pl.ds(s * GPT, GPT)])
        pltpu.sync_copy(hist_v.at[pl.ds(0, G)], ocnt.at[pl.ds(wid * G, G)])

    return k(x, ids)


def _combine(psum, pcnt):
    def body(ps_ref, pc_ref, o_ref):
        ps = ps_ref[...]
        cnt = jnp.sum(pc_ref[...].reshape(NW, G), axis=0).astype(jnp.float32)
        cnt = jnp.maximum(cnt, 1.0)
        o_ref[...] = (ps[0] + ps[1]) / cnt[:, None]

    return pl.pallas_call(
        body,
        out_shape=jax.ShapeDtypeStruct((G, D), jnp.float32),
    )(psum, pcnt)


def kernel(input, batch, num_graphs):
    ids = batch.astype(jnp.int32)
    psum, pcnt = _sc_partials(input, ids)
    return _combine(psum, pcnt)
